# baseline (speedup 1.0000x reference)
import functools

import jax
import jax.numpy as jnp
from jax import lax
from jax.experimental import pallas as pl
from jax.experimental.pallas import tpu as pltpu
from jax.experimental.pallas import tpu_sc as plsc

CTX_DIM = 512
N_CLS_CTX = 4
N_PRE = 5
TOK_LEN = 77
N_SUF = TOK_LEN - N_PRE - N_CLS_CTX
ROW = N_CLS_CTX * CTX_DIM


def _sc_gather(table2d, label):
    info = plsc.get_sparse_core_info()
    num_workers = info.num_cores * info.num_subcores
    b = label.shape[0]
    assert b % num_workers == 0
    bpw = b // num_workers

    mesh = plsc.VectorSubcoreMesh(core_axis_name="c", subcore_axis_name="s")

    @functools.partial(
        pl.kernel,
        mesh=mesh,
        out_type=jax.ShapeDtypeStruct((b, ROW), jnp.float32),
        scratch_types=[
            pltpu.VMEM((bpw,), jnp.int32),
            pltpu.VMEM((bpw, ROW), jnp.float32),
            pltpu.SemaphoreType.DMA,
        ],
    )
    def gather_kernel(table_hbm, idx_hbm, out_hbm, idx_v, rows_v, sem):
        wid = lax.axis_index("s") * info.num_cores + lax.axis_index("c")
        base = wid * bpw
        pltpu.sync_copy(idx_hbm.at[pl.ds(base, bpw)], idx_v)
        pltpu.async_copy(table_hbm.at[idx_v], rows_v, sem).wait()
        pltpu.sync_copy(rows_v, out_hbm.at[pl.ds(base, bpw)])

    return gather_kernel(table2d, label)


def _tc_concat(cls3, token_prefix, token_suffix, block_b=8):
    b = cls3.shape[0]
    assert b % block_b == 0

    def body(pref_ref, suf_ref, cls_ref, out_ref):
        out_ref[:, 0:N_PRE, :] = jnp.broadcast_to(
            pref_ref[...], (block_b, N_PRE, CTX_DIM))
        out_ref[:, N_PRE:N_PRE + N_CLS_CTX, :] = cls_ref[...]
        out_ref[:, N_PRE + N_CLS_CTX:TOK_LEN, :] = jnp.broadcast_to(
            suf_ref[...], (block_b, N_SUF, CTX_DIM))

    return pl.pallas_call(
        body,
        grid=(b // block_b,),
        in_specs=[
            pl.BlockSpec((1, N_PRE, CTX_DIM), lambda i: (0, 0, 0)),
            pl.BlockSpec((1, N_SUF, CTX_DIM), lambda i: (0, 0, 0)),
            pl.BlockSpec((block_b, N_CLS_CTX, CTX_DIM), lambda i: (i, 0, 0)),
        ],
        out_specs=pl.BlockSpec((block_b, TOK_LEN, CTX_DIM), lambda i: (i, 0, 0)),
        out_shape=jax.ShapeDtypeStruct((b, TOK_LEN, CTX_DIM), jnp.float32),
    )(token_prefix, token_suffix, cls3)


def kernel(label, cls_ctx, token_prefix, token_suffix):
    num_class = cls_ctx.shape[0]
    table2d = cls_ctx.reshape(num_class, ROW)
    cls2 = _sc_gather(table2d, label.astype(jnp.int32))
    cls3 = cls2.reshape(-1, N_CLS_CTX, CTX_DIM)
    return _tc_concat(cls3, token_prefix, token_suffix)


# speedup vs baseline: 3.6480x; 3.6480x over previous
import functools

import jax
import jax.numpy as jnp
from jax import lax
from jax.experimental import pallas as pl
from jax.experimental.pallas import tpu as pltpu
from jax.experimental.pallas import tpu_sc as plsc

CTX_DIM = 512
N_CLS_CTX = 4
N_PRE = 5
TOK_LEN = 77
N_SUF = TOK_LEN - N_PRE - N_CLS_CTX
ROW = N_CLS_CTX * CTX_DIM


def _sc_gather(table3d, label):
    info = plsc.get_sparse_core_info()
    num_workers = info.num_cores * info.num_subcores
    b = label.shape[0]
    assert b % num_workers == 0
    bpw = b // num_workers

    mesh = plsc.VectorSubcoreMesh(core_axis_name="c", subcore_axis_name="s")

    @functools.partial(
        pl.kernel,
        mesh=mesh,
        out_type=jax.ShapeDtypeStruct((b, N_CLS_CTX, CTX_DIM), jnp.float32),
        scratch_types=[
            pltpu.VMEM((bpw,), jnp.int32),
            pltpu.VMEM((bpw, N_CLS_CTX, CTX_DIM), jnp.float32),
            pltpu.SemaphoreType.DMA,
        ],
    )
    def gather_kernel(table_hbm, idx_hbm, out_hbm, idx_v, rows_v, sem):
        wid = lax.axis_index("s") * info.num_cores + lax.axis_index("c")
        base = wid * bpw
        pltpu.sync_copy(idx_hbm.at[pl.ds(base, bpw)], idx_v)
        pltpu.async_copy(table_hbm.at[idx_v], rows_v, sem).wait()
        pltpu.sync_copy(rows_v, out_hbm.at[pl.ds(base, bpw)])

    return gather_kernel(table3d, label)


def _tc_concat(cls3, token_prefix, token_suffix, block_b=8):
    b = cls3.shape[0]
    assert b % block_b == 0

    def body(pref_ref, suf_ref, cls_ref, out_ref):
        out_ref[:, 0:N_PRE, :] = jnp.broadcast_to(
            pref_ref[...], (block_b, N_PRE, CTX_DIM))
        out_ref[:, N_PRE:N_PRE + N_CLS_CTX, :] = cls_ref[...]
        out_ref[:, N_PRE + N_CLS_CTX:TOK_LEN, :] = jnp.broadcast_to(
            suf_ref[...], (block_b, N_SUF, CTX_DIM))

    return pl.pallas_call(
        body,
        grid=(b // block_b,),
        in_specs=[
            pl.BlockSpec((1, N_PRE, CTX_DIM), lambda i: (0, 0, 0)),
            pl.BlockSpec((1, N_SUF, CTX_DIM), lambda i: (0, 0, 0)),
            pl.BlockSpec((block_b, N_CLS_CTX, CTX_DIM), lambda i: (i, 0, 0)),
        ],
        out_specs=pl.BlockSpec((block_b, TOK_LEN, CTX_DIM), lambda i: (i, 0, 0)),
        out_shape=jax.ShapeDtypeStruct((b, TOK_LEN, CTX_DIM), jnp.float32),
    )(token_prefix, token_suffix, cls3)


def kernel(label, cls_ctx, token_prefix, token_suffix):
    cls3 = _sc_gather(cls_ctx, label.astype(jnp.int32))
    return _tc_concat(cls3, token_prefix, token_suffix)


# speedup vs baseline: 4.4626x; 1.2233x over previous
import functools

import jax
import jax.numpy as jnp
from jax import lax
from jax.experimental import pallas as pl
from jax.experimental.pallas import tpu as pltpu
from jax.experimental.pallas import tpu_sc as plsc

CTX_DIM = 512
N_CLS_CTX = 4
N_PRE = 5
TOK_LEN = 77
N_SUF = TOK_LEN - N_PRE - N_CLS_CTX
ROW = N_CLS_CTX * CTX_DIM


def _sc_gather(table3d, label):
    info = plsc.get_sparse_core_info()
    num_workers = info.num_cores * info.num_subcores
    b = label.shape[0]
    assert b % num_workers == 0
    bpw = b // num_workers

    mesh = plsc.VectorSubcoreMesh(core_axis_name="c", subcore_axis_name="s")

    @functools.partial(
        pl.kernel,
        mesh=mesh,
        out_type=jax.ShapeDtypeStruct((b, N_CLS_CTX, CTX_DIM), jnp.float32),
        scratch_types=[
            pltpu.VMEM((bpw,), jnp.int32),
            pltpu.VMEM((bpw, N_CLS_CTX, CTX_DIM), jnp.float32),
            pltpu.SemaphoreType.DMA,
        ],
    )
    def gather_kernel(table_hbm, idx_hbm, out_hbm, idx_v, rows_v, sem):
        wid = lax.axis_index("s") * info.num_cores + lax.axis_index("c")
        base = wid * bpw
        pltpu.sync_copy(idx_hbm.at[pl.ds(base, bpw)], idx_v)
        pltpu.async_copy(table_hbm.at[idx_v], rows_v, sem).wait()
        pltpu.sync_copy(rows_v, out_hbm.at[pl.ds(base, bpw)])

    return gather_kernel(table3d, label)


def _tc_concat(cls3, token_prefix, token_suffix, block_b=32):
    b = cls3.shape[0]
    assert b % block_b == 0

    def body(pref_ref, suf_ref, cls_ref, out_ref):
        out_ref[:, 0:N_PRE, :] = jnp.broadcast_to(
            pref_ref[...], (block_b, N_PRE, CTX_DIM))
        out_ref[:, N_PRE:N_PRE + N_CLS_CTX, :] = cls_ref[...]
        out_ref[:, N_PRE + N_CLS_CTX:TOK_LEN, :] = jnp.broadcast_to(
            suf_ref[...], (block_b, N_SUF, CTX_DIM))

    return pl.pallas_call(
        body,
        grid=(b // block_b,),
        in_specs=[
            pl.BlockSpec((1, N_PRE, CTX_DIM), lambda i: (0, 0, 0)),
            pl.BlockSpec((1, N_SUF, CTX_DIM), lambda i: (0, 0, 0)),
            pl.BlockSpec((block_b, N_CLS_CTX, CTX_DIM), lambda i: (i, 0, 0)),
        ],
        out_specs=pl.BlockSpec((block_b, TOK_LEN, CTX_DIM), lambda i: (i, 0, 0)),
        out_shape=jax.ShapeDtypeStruct((b, TOK_LEN, CTX_DIM), jnp.float32),
    )(token_prefix, token_suffix, cls3)


def kernel(label, cls_ctx, token_prefix, token_suffix):
    cls3 = _sc_gather(cls_ctx, label.astype(jnp.int32))
    return _tc_concat(cls3, token_prefix, token_suffix)


# speedup vs baseline: 4.5095x; 1.0105x over previous
import functools

import jax
import jax.numpy as jnp
from jax import lax
from jax.experimental import pallas as pl
from jax.experimental.pallas import tpu as pltpu
from jax.experimental.pallas import tpu_sc as plsc

CTX_DIM = 512
N_CLS_CTX = 4
N_PRE = 5
TOK_LEN = 77
N_SUF = TOK_LEN - N_PRE - N_CLS_CTX
ROW = N_CLS_CTX * CTX_DIM


def _sc_gather(table3d, label):
    info = plsc.get_sparse_core_info()
    num_workers = info.num_cores * info.num_subcores
    b = label.shape[0]
    assert b % num_workers == 0
    bpw = b // num_workers

    mesh = plsc.VectorSubcoreMesh(core_axis_name="c", subcore_axis_name="s")

    @functools.partial(
        pl.kernel,
        mesh=mesh,
        out_type=jax.ShapeDtypeStruct((b, N_CLS_CTX, CTX_DIM), jnp.float32),
        scratch_types=[
            pltpu.VMEM((bpw,), jnp.int32),
            pltpu.VMEM((bpw, N_CLS_CTX, CTX_DIM), jnp.float32),
            pltpu.SemaphoreType.DMA,
        ],
    )
    def gather_kernel(table_hbm, idx_hbm, out_hbm, idx_v, rows_v, sem):
        wid = lax.axis_index("s") * info.num_cores + lax.axis_index("c")
        base = wid * bpw
        pltpu.sync_copy(idx_hbm.at[pl.ds(base, bpw)], idx_v)
        pltpu.async_copy(table_hbm.at[idx_v], rows_v, sem).wait()
        pltpu.sync_copy(rows_v, out_hbm.at[pl.ds(base, bpw)])

    return gather_kernel(table3d, label)


def _tc_concat(cls3, token_prefix, token_suffix, block_b=64):
    b = cls3.shape[0]
    assert b % block_b == 0

    def body(pref_ref, suf_ref, cls_ref, out_ref):
        out_ref[:, 0:N_PRE, :] = jnp.broadcast_to(
            pref_ref[...], (block_b, N_PRE, CTX_DIM))
        out_ref[:, N_PRE:N_PRE + N_CLS_CTX, :] = cls_ref[...]
        out_ref[:, N_PRE + N_CLS_CTX:TOK_LEN, :] = jnp.broadcast_to(
            suf_ref[...], (block_b, N_SUF, CTX_DIM))

    return pl.pallas_call(
        body,
        grid=(b // block_b,),
        in_specs=[
            pl.BlockSpec((1, N_PRE, CTX_DIM), lambda i: (0, 0, 0)),
            pl.BlockSpec((1, N_SUF, CTX_DIM), lambda i: (0, 0, 0)),
            pl.BlockSpec((block_b, N_CLS_CTX, CTX_DIM), lambda i: (i, 0, 0)),
        ],
        out_specs=pl.BlockSpec((block_b, TOK_LEN, CTX_DIM), lambda i: (i, 0, 0)),
        out_shape=jax.ShapeDtypeStruct((b, TOK_LEN, CTX_DIM), jnp.float32),
    )(token_prefix, token_suffix, cls3)


def kernel(label, cls_ctx, token_prefix, token_suffix):
    cls3 = _sc_gather(cls_ctx, label.astype(jnp.int32))
    return _tc_concat(cls3, token_prefix, token_suffix)
